# baseline (speedup 1.0000x reference)
import jax
import jax.numpy as jnp
from jax import lax
from jax.experimental import pallas as pl
from jax.experimental.pallas import tpu as pltpu
from jax.experimental.pallas import tpu_sc as plsc

N = 100000
NH = N // 2
ATT = 16
EMB = 128
OUT = 3 * ATT + EMB
APAD = 128
L = 16

VA, VG, VC = 10, 3, 100

NC = 2
NS = 16
NW = NC * NS

BLK = 1024
NBLK = NH // BLK
REM = NH - NBLK * BLK

NSLOT = 2


def _sc_body(age_hbm, gen_hbm, city_hbm, wa_hbm, wg_hbm, wc_hbm,
             att_hbm, ia, ig, ic, ra, rg, rc, wa_v, wg_v, wc_v, sa, sc_):
  wid = lax.axis_index("s") * NC + lax.axis_index("c")

  pltpu.sync_copy(wa_hbm, wa_v)
  pltpu.sync_copy(wg_hbm, wg_v)
  pltpu.sync_copy(wc_hbm, wc_v)

  def issue_idx(base, n, s):
    pltpu.async_copy(age_hbm.at[pl.ds(base, n)], ia.at[s, pl.ds(0, n)], sa)
    pltpu.async_copy(gen_hbm.at[pl.ds(base, n)], ig.at[s, pl.ds(0, n)], sa)
    pltpu.async_copy(city_hbm.at[pl.ds(base, n)], ic.at[s, pl.ds(0, n)], sa)

  def wait_idx(n, s):
    pltpu.make_async_copy(age_hbm.at[pl.ds(0, n)], ia.at[s, pl.ds(0, n)], sa).wait()
    pltpu.make_async_copy(gen_hbm.at[pl.ds(0, n)], ig.at[s, pl.ds(0, n)], sa).wait()
    pltpu.make_async_copy(city_hbm.at[pl.ds(0, n)], ic.at[s, pl.ds(0, n)], sa).wait()

  def gather_block(n, s):
    lanes = lax.iota(jnp.int32, L)

    @pl.loop(0, n // L, unroll=4)
    def group(k):
      rowpos = k * L + lanes
      a_rows = ia[s, pl.ds(k * L, L)]
      g_rows = ig[s, pl.ds(k * L, L)]
      c_rows = ic[s, pl.ds(k * L, L)]
      for d in range(ATT):
        cols = jnp.full((L,), d, jnp.int32)
        plsc.store_scatter(ra.at[s], [rowpos, cols],
                           plsc.load_gather(wa_v, [a_rows, cols]))
        plsc.store_scatter(rg.at[s], [rowpos, cols],
                           plsc.load_gather(wg_v, [g_rows, cols]))
        plsc.store_scatter(rc.at[s], [rowpos, cols],
                           plsc.load_gather(wc_v, [c_rows, cols]))

  def issue_writes(base, n, s):
    pltpu.async_copy(ra.at[s, pl.ds(0, n)],
                     att_hbm.at[pl.ds(base, n), pl.ds(0, ATT)], sc_)
    pltpu.async_copy(rg.at[s, pl.ds(0, n)],
                     att_hbm.at[pl.ds(base, n), pl.ds(ATT, ATT)], sc_)
    pltpu.async_copy(rc.at[s, pl.ds(0, n)],
                     att_hbm.at[pl.ds(base, n), pl.ds(2 * ATT, ATT)], sc_)

  def wait_writes(n, s):
    pltpu.make_async_copy(ra.at[s, pl.ds(0, n)],
                          att_hbm.at[pl.ds(0, n), pl.ds(0, ATT)], sc_).wait()
    pltpu.make_async_copy(rg.at[s, pl.ds(0, n)],
                          att_hbm.at[pl.ds(0, n), pl.ds(ATT, ATT)], sc_).wait()
    pltpu.make_async_copy(rc.at[s, pl.ds(0, n)],
                          att_hbm.at[pl.ds(0, n), pl.ds(2 * ATT, ATT)], sc_).wait()

  @pl.loop(wid, NBLK, step=NW * NSLOT)
  def block(b):
    for s in range(NSLOT):
      @pl.when(b + s * NW < NBLK)
      def _():
        issue_idx((b + s * NW) * BLK, BLK, s)
    for s in range(NSLOT):
      @pl.when(b + s * NW < NBLK)
      def _():
        wait_idx(BLK, s)
        gather_block(BLK, s)
        issue_writes((b + s * NW) * BLK, BLK, s)
    for s in range(NSLOT):
      @pl.when(b + s * NW < NBLK)
      def _():
        wait_writes(BLK, s)

  if REM:
    @pl.when(wid == NW - 1)
    def _tail():
      base = NBLK * BLK
      issue_idx(base, REM, 0)
      wait_idx(REM, 0)
      gather_block(REM, 0)
      issue_writes(base, REM, 0)
      wait_writes(REM, 0)


def _sc_gather(age_idx, gender_idx, city_idx, W_age, W_gender, W_city):
  mesh = plsc.VectorSubcoreMesh(core_axis_name="c", subcore_axis_name="s")
  f = pl.kernel(
      _sc_body,
      out_type=jax.ShapeDtypeStruct((NH, APAD), jnp.float32),
      mesh=mesh,
      compiler_params=pltpu.CompilerParams(
          use_tc_tiling_on_sc=False, needs_layout_passes=False),
      scratch_types=[
          pltpu.VMEM((NSLOT, BLK), jnp.int32),
          pltpu.VMEM((NSLOT, BLK), jnp.int32),
          pltpu.VMEM((NSLOT, BLK), jnp.int32),
          pltpu.VMEM((NSLOT, BLK, ATT), jnp.float32),
          pltpu.VMEM((NSLOT, BLK, ATT), jnp.float32),
          pltpu.VMEM((NSLOT, BLK, ATT), jnp.float32),
          pltpu.VMEM((VA, ATT), jnp.float32),
          pltpu.VMEM((VG, ATT), jnp.float32),
          pltpu.VMEM((VC, ATT), jnp.float32),
          pltpu.SemaphoreType.DMA,
          pltpu.SemaphoreType.DMA,
      ],
  )
  return f(age_idx, gender_idx, city_idx, W_age, W_gender, W_city)


B = 5000
HB = NH // B


def _tc_body_first(att_ref, emb_ref, out_ref):
  out_ref[:, :] = jnp.concatenate(
      [att_ref[:, : 3 * ATT], emb_ref[:, :]], axis=1)


def _tc_body_second(att_ref, emb_ref, prev_ref, out_ref):
  del prev_ref
  out_ref[:, :] = jnp.concatenate(
      [att_ref[:, : 3 * ATT], emb_ref[:, :]], axis=1)


def _tc_concat_first(att_lo, emb_feat):
  return pl.pallas_call(
      _tc_body_first,
      grid=(HB,),
      in_specs=[
          pl.BlockSpec((B, APAD), lambda i: (i, 0)),
          pl.BlockSpec((B, EMB), lambda i: (i, 0)),
      ],
      out_specs=pl.BlockSpec((B, OUT), lambda i: (i, 0)),
      out_shape=jax.ShapeDtypeStruct((N, OUT), jnp.float32),
  )(att_lo, emb_feat)


def _tc_concat_second(att_hi, emb_feat, prev):
  return pl.pallas_call(
      _tc_body_second,
      grid=(HB,),
      in_specs=[
          pl.BlockSpec((B, APAD), lambda i: (i, 0)),
          pl.BlockSpec((B, EMB), lambda i: (i + HB, 0)),
          pl.BlockSpec(memory_space=pl.ANY),
      ],
      out_specs=pl.BlockSpec((B, OUT), lambda i: (i + HB, 0)),
      out_shape=jax.ShapeDtypeStruct((N, OUT), jnp.float32),
      input_output_aliases={2: 0},
  )(att_hi, emb_feat, prev)


@jax.jit
def _run(age_idx, gender_idx, city_idx, emb_feat, W_age, W_gender, W_city):
  att_lo = _sc_gather(age_idx[:NH], gender_idx[:NH], city_idx[:NH],
                      W_age, W_gender, W_city)
  att_hi = _sc_gather(age_idx[NH:], gender_idx[NH:], city_idx[NH:],
                      W_age, W_gender, W_city)
  out = _tc_concat_first(att_lo, emb_feat)
  return _tc_concat_second(att_hi, emb_feat, out)


def kernel(age_idx, gender_idx, city_idx, emb_feat, W_age, W_gender, W_city):
  return _run(
      age_idx.astype(jnp.int32),
      gender_idx.astype(jnp.int32),
      city_idx.astype(jnp.int32),
      emb_feat, W_age, W_gender, W_city)


# speedup vs baseline: 1.0320x; 1.0320x over previous
import jax
import jax.numpy as jnp
from jax import lax
from jax.experimental import pallas as pl
from jax.experimental.pallas import tpu as pltpu
from jax.experimental.pallas import tpu_sc as plsc

N = 100000
NH = N // 2
ATT = 16
EMB = 128
OUT = 3 * ATT + EMB
APAD = 128
L = 16

VA, VG, VC = 10, 3, 100

NC = 2
NS = 16
NW = NC * NS

BLK = 1552
NBLK = NH // BLK
REM = NH - NBLK * BLK

NSLOT = 1


def _sc_body(age_hbm, gen_hbm, city_hbm, wa_hbm, wg_hbm, wc_hbm,
             att_hbm, ia, ig, ic, ra, rg, rc, wa_v, wg_v, wc_v, sa, sc_):
  wid = lax.axis_index("s") * NC + lax.axis_index("c")

  pltpu.sync_copy(wa_hbm, wa_v)
  pltpu.sync_copy(wg_hbm, wg_v)
  pltpu.sync_copy(wc_hbm, wc_v)

  def issue_idx(base, n, s):
    pltpu.async_copy(age_hbm.at[pl.ds(base, n)], ia.at[s, pl.ds(0, n)], sa)
    pltpu.async_copy(gen_hbm.at[pl.ds(base, n)], ig.at[s, pl.ds(0, n)], sa)
    pltpu.async_copy(city_hbm.at[pl.ds(base, n)], ic.at[s, pl.ds(0, n)], sa)

  def wait_idx(n, s):
    pltpu.make_async_copy(age_hbm.at[pl.ds(0, n)], ia.at[s, pl.ds(0, n)], sa).wait()
    pltpu.make_async_copy(gen_hbm.at[pl.ds(0, n)], ig.at[s, pl.ds(0, n)], sa).wait()
    pltpu.make_async_copy(city_hbm.at[pl.ds(0, n)], ic.at[s, pl.ds(0, n)], sa).wait()

  def gather_block(n, s):
    lanes = lax.iota(jnp.int32, L)

    @pl.loop(0, n // L)
    def group(k):
      rowpos = k * L + lanes
      a_rows = ia[s, pl.ds(k * L, L)]
      g_rows = ig[s, pl.ds(k * L, L)]
      c_rows = ic[s, pl.ds(k * L, L)]
      for d in range(ATT):
        cols = jnp.full((L,), d, jnp.int32)
        plsc.store_scatter(ra.at[s], [rowpos, cols],
                           plsc.load_gather(wa_v, [a_rows, cols]))
        plsc.store_scatter(rg.at[s], [rowpos, cols],
                           plsc.load_gather(wg_v, [g_rows, cols]))
        plsc.store_scatter(rc.at[s], [rowpos, cols],
                           plsc.load_gather(wc_v, [c_rows, cols]))

  def issue_writes(base, n, s):
    pltpu.async_copy(ra.at[s, pl.ds(0, n)],
                     att_hbm.at[pl.ds(base, n), pl.ds(0, ATT)], sc_)
    pltpu.async_copy(rg.at[s, pl.ds(0, n)],
                     att_hbm.at[pl.ds(base, n), pl.ds(ATT, ATT)], sc_)
    pltpu.async_copy(rc.at[s, pl.ds(0, n)],
                     att_hbm.at[pl.ds(base, n), pl.ds(2 * ATT, ATT)], sc_)

  def wait_writes(n, s):
    pltpu.make_async_copy(ra.at[s, pl.ds(0, n)],
                          att_hbm.at[pl.ds(0, n), pl.ds(0, ATT)], sc_).wait()
    pltpu.make_async_copy(rg.at[s, pl.ds(0, n)],
                          att_hbm.at[pl.ds(0, n), pl.ds(ATT, ATT)], sc_).wait()
    pltpu.make_async_copy(rc.at[s, pl.ds(0, n)],
                          att_hbm.at[pl.ds(0, n), pl.ds(2 * ATT, ATT)], sc_).wait()

  @pl.loop(wid, NBLK, step=NW * NSLOT)
  def block(b):
    for s in range(NSLOT):
      @pl.when(b + s * NW < NBLK)
      def _():
        issue_idx((b + s * NW) * BLK, BLK, s)
    for s in range(NSLOT):
      @pl.when(b + s * NW < NBLK)
      def _():
        wait_idx(BLK, s)
        gather_block(BLK, s)
        issue_writes((b + s * NW) * BLK, BLK, s)
    for s in range(NSLOT):
      @pl.when(b + s * NW < NBLK)
      def _():
        wait_writes(BLK, s)

  if REM:
    @pl.when(wid == NW - 1)
    def _tail():
      base = NBLK * BLK
      issue_idx(base, REM, 0)
      wait_idx(REM, 0)
      gather_block(REM, 0)
      issue_writes(base, REM, 0)
      wait_writes(REM, 0)


def _sc_gather(age_idx, gender_idx, city_idx, W_age, W_gender, W_city):
  mesh = plsc.VectorSubcoreMesh(core_axis_name="c", subcore_axis_name="s")
  f = pl.kernel(
      _sc_body,
      out_type=jax.ShapeDtypeStruct((NH, APAD), jnp.float32),
      mesh=mesh,
      compiler_params=pltpu.CompilerParams(
          use_tc_tiling_on_sc=False, needs_layout_passes=False),
      scratch_types=[
          pltpu.VMEM((NSLOT, BLK), jnp.int32),
          pltpu.VMEM((NSLOT, BLK), jnp.int32),
          pltpu.VMEM((NSLOT, BLK), jnp.int32),
          pltpu.VMEM((NSLOT, BLK, ATT), jnp.float32),
          pltpu.VMEM((NSLOT, BLK, ATT), jnp.float32),
          pltpu.VMEM((NSLOT, BLK, ATT), jnp.float32),
          pltpu.VMEM((VA, ATT), jnp.float32),
          pltpu.VMEM((VG, ATT), jnp.float32),
          pltpu.VMEM((VC, ATT), jnp.float32),
          pltpu.SemaphoreType.DMA,
          pltpu.SemaphoreType.DMA,
      ],
  )
  return f(age_idx, gender_idx, city_idx, W_age, W_gender, W_city)


B = 5000
HB = NH // B


def _tc_body_first(att_ref, emb_ref, out_ref):
  out_ref[:, :] = jnp.concatenate(
      [att_ref[:, : 3 * ATT], emb_ref[:, :]], axis=1)


def _tc_body_second(att_ref, emb_ref, prev_ref, out_ref):
  del prev_ref
  out_ref[:, :] = jnp.concatenate(
      [att_ref[:, : 3 * ATT], emb_ref[:, :]], axis=1)


def _tc_concat_first(att_lo, emb_feat):
  return pl.pallas_call(
      _tc_body_first,
      grid=(HB,),
      in_specs=[
          pl.BlockSpec((B, APAD), lambda i: (i, 0)),
          pl.BlockSpec((B, EMB), lambda i: (i, 0)),
      ],
      out_specs=pl.BlockSpec((B, OUT), lambda i: (i, 0)),
      out_shape=jax.ShapeDtypeStruct((N, OUT), jnp.float32),
  )(att_lo, emb_feat)


def _tc_concat_second(att_hi, emb_feat, prev):
  return pl.pallas_call(
      _tc_body_second,
      grid=(HB,),
      in_specs=[
          pl.BlockSpec((B, APAD), lambda i: (i, 0)),
          pl.BlockSpec((B, EMB), lambda i: (i + HB, 0)),
          pl.BlockSpec(memory_space=pl.ANY),
      ],
      out_specs=pl.BlockSpec((B, OUT), lambda i: (i + HB, 0)),
      out_shape=jax.ShapeDtypeStruct((N, OUT), jnp.float32),
      input_output_aliases={2: 0},
  )(att_hi, emb_feat, prev)


@jax.jit
def _run(age_idx, gender_idx, city_idx, emb_feat, W_age, W_gender, W_city):
  att_lo = _sc_gather(age_idx[:NH], gender_idx[:NH], city_idx[:NH],
                      W_age, W_gender, W_city)
  att_hi = _sc_gather(age_idx[NH:], gender_idx[NH:], city_idx[NH:],
                      W_age, W_gender, W_city)
  out = _tc_concat_first(att_lo, emb_feat)
  return _tc_concat_second(att_hi, emb_feat, out)


def kernel(age_idx, gender_idx, city_idx, emb_feat, W_age, W_gender, W_city):
  return _run(
      age_idx.astype(jnp.int32),
      gender_idx.astype(jnp.int32),
      city_idx.astype(jnp.int32),
      emb_feat, W_age, W_gender, W_city)
